# baseline (speedup 1.0000x reference)
import jax
import jax.numpy as jnp
from jax import lax
from jax.experimental import pallas as pl
from jax.experimental.pallas import tpu as pltpu
from jax.experimental.pallas import tpu_sc as plsc

N = 10000
E = 320000
D = 128
H = 128
NUM_REL = 4
NCLS = 16
METAPATHS = ((0, 1), (2, 3), (1, 0))

NC = 2
NS = 16
NW = NC * NS
LANES = 16
CHUNK = E // NW
VPC = CHUNK // LANES
NPAD = 10240
TRASH = N
BLK = 128
NBLK = 81
CAP = NBLK * BLK
SLAB = NPAD // NS

BROWS = 1000
GRID = N // BROWS


def _mesh():
    return plsc.VectorSubcoreMesh(core_axis_name="c", subcore_axis_name="s",
                                  num_cores=NC, num_subcores=NS)


_SC_PARAMS = pltpu.CompilerParams(needs_layout_passes=False,
                                  use_tc_tiling_on_sc=False)


def _wid():
    return lax.axis_index("s") * NC + lax.axis_index("c")


def _bucket_body(src_hbm, dst_hbm, type_hbm, bsrc_hbm, bdst_hbm, cnt_hbm,
                 src_v, dst_v, type_v, bsrc_v, bdst_v, cnt_v):
    wid = _wid()
    base = wid * CHUNK
    pltpu.sync_copy(src_hbm.at[pl.ds(base, CHUNK)], src_v)
    pltpu.sync_copy(dst_hbm.at[pl.ds(base, CHUNK)], dst_v)
    pltpu.sync_copy(type_hbm.at[pl.ds(base, CHUNK)], type_v)

    trash = jnp.full((LANES,), TRASH, jnp.int32)
    zero = jnp.zeros((LANES,), jnp.int32)

    def prefill(i, _):
        for r in range(NUM_REL):
            bsrc_v[pl.ds(r * CAP + i * LANES, LANES)] = trash
            bdst_v[pl.ds(r * CAP + i * LANES, LANES)] = zero
        return 0

    lax.fori_loop(0, CAP // LANES, prefill, 0)

    one = jnp.ones((LANES,), jnp.int32)

    def step(i, offs):
        s = src_v[pl.ds(i * LANES, LANES)]
        d = dst_v[pl.ds(i * LANES, LANES)]
        t = type_v[pl.ds(i * LANES, LANES)]
        new = []
        for r in range(NUM_REL):
            m = t == r
            c = plsc.cumsum(jnp.where(m, one, zero))
            pos = offs[r] + c - 1 + r * CAP
            plsc.store_scatter(bsrc_v, [pos], s, mask=m)
            plsc.store_scatter(bdst_v, [pos], d, mask=m)
            new.append(offs[r] + plsc.all_reduce_population_count(m))
        return tuple(new)

    offs = lax.fori_loop(0, VPC, step,
                         tuple(jnp.zeros((LANES,), jnp.int32)
                               for _ in range(NUM_REL)))
    for r in range(NUM_REL):
        cnt_v[pl.ds(r * LANES, LANES)] = offs[r]
        pltpu.sync_copy(bsrc_v.at[pl.ds(r * CAP, CAP)], bsrc_hbm.at[r, wid])
        pltpu.sync_copy(bdst_v.at[pl.ds(r * CAP, CAP)], bdst_hbm.at[r, wid])
    pltpu.sync_copy(cnt_v, cnt_hbm.at[wid])


def _bucket_call(src, dst, etype):
    k = pl.kernel(
        _bucket_body,
        out_type=(
            jax.ShapeDtypeStruct((NUM_REL, NW, CAP), jnp.int32),
            jax.ShapeDtypeStruct((NUM_REL, NW, CAP), jnp.int32),
            jax.ShapeDtypeStruct((NW, NUM_REL * LANES), jnp.int32),
        ),
        mesh=_mesh(),
        compiler_params=_SC_PARAMS,
        scratch_types=[
            pltpu.VMEM((CHUNK,), jnp.int32),
            pltpu.VMEM((CHUNK,), jnp.int32),
            pltpu.VMEM((CHUNK,), jnp.int32),
            pltpu.VMEM((NUM_REL * CAP,), jnp.int32),
            pltpu.VMEM((NUM_REL * CAP,), jnp.int32),
            pltpu.VMEM((NUM_REL * LANES,), jnp.int32),
        ],
    )
    return k(src, dst, etype)


def _segsum_body(h_hbm, bsrc_hbm, bdst_hbm, cnt_hbm, out_hbm, degp_hbm,
                 idx_s2, idx_d2, rows_a, rows_b, cnt_v, deg_v,
                 accum, sem_a, sem_b):
    cid = lax.axis_index("c")
    sid = lax.axis_index("s")
    wid = sid * NC + cid

    zero = jnp.zeros((LANES,), jnp.float32)

    def zfill(i, _):
        for j in range(D // LANES):
            rows_a[i, pl.ds(j * LANES, LANES)] = zero
        return 0

    lax.fori_loop(0, BLK, zfill, 0)
    for k in range(SLAB // BLK):
        pltpu.sync_copy(rows_a, accum.at[pl.ds(sid * SLAB + k * BLK, BLK)])

    def dzfill(i, _):
        deg_v[pl.ds(i * LANES, LANES)] = zero
        return 0

    lax.fori_loop(0, NPAD // LANES, dzfill, 0)
    plsc.subcore_barrier()

    pltpu.sync_copy(cnt_hbm.at[wid], cnt_v)
    n = jnp.max(cnt_v[...])
    nblk = jnp.maximum((n + BLK - 1) >> 7, 1)
    nblk2 = (nblk + 1) >> 1

    ones = jnp.ones((LANES,), jnp.float32)

    def loop(i, _):
        b0 = 2 * i
        pltpu.sync_copy(bsrc_hbm.at[wid, pl.ds(b0, 2)], idx_s2)
        pltpu.sync_copy(bdst_hbm.at[wid, pl.ds(b0, 2)], idx_d2)
        pltpu.async_copy(h_hbm.at[idx_d2.at[0]], rows_a, sem_a)
        pltpu.async_copy(h_hbm.at[idx_d2.at[1]], rows_b, sem_b)
        for half in range(2):
            for j in range(BLK // LANES):
                s = idx_s2[half, pl.ds(j * LANES, LANES)]
                plsc.addupdate_scatter(deg_v, [s], ones)
        pltpu.make_async_copy(h_hbm.at[idx_d2.at[0]], rows_a, sem_a).wait()
        pltpu.sync_copy(rows_a, accum.at[idx_s2.at[0]], add=True)
        pltpu.make_async_copy(h_hbm.at[idx_d2.at[1]], rows_b, sem_b).wait()
        pltpu.sync_copy(rows_b, accum.at[idx_s2.at[1]], add=True)
        return 0

    lax.fori_loop(0, nblk2, loop, 0)

    plsc.subcore_barrier()
    for k in range(SLAB // BLK):
        sl = pl.ds(sid * SLAB + k * BLK, BLK)
        pltpu.sync_copy(accum.at[sl], out_hbm.at[cid, sl])
    pltpu.sync_copy(deg_v, degp_hbm.at[wid])


def _segsum_call(h, bsrc_r, bdst_r, cnt_r):
    k = pl.kernel(
        _segsum_body,
        out_type=(
            jax.ShapeDtypeStruct((NC, NPAD, D), jnp.float32),
            jax.ShapeDtypeStruct((NW, NPAD), jnp.float32),
        ),
        mesh=_mesh(),
        compiler_params=_SC_PARAMS,
        scratch_types=[
            pltpu.VMEM((2, BLK), jnp.int32),
            pltpu.VMEM((2, BLK), jnp.int32),
            pltpu.VMEM((BLK, D), jnp.float32),
            pltpu.VMEM((BLK, D), jnp.float32),
            pltpu.VMEM((LANES,), jnp.int32),
            pltpu.VMEM((NPAD,), jnp.float32),
            pltpu.VMEM_SHARED((NPAD, D), jnp.float32),
            pltpu.SemaphoreType.DMA,
            pltpu.SemaphoreType.DMA,
        ],
    )
    return k(h, bsrc_r, bdst_r, cnt_r)


def _dense1_body(x_ref, sp0, sp1, sp2, dg0, dg1, dg2,
                 wr0, wt0, b0, wr1, wt1, b1, wr2, wt2, b2,
                 h0, h1, h2):
    x = x_ref[...]
    for sp, dg, wr, wt, bb, out in ((sp0, dg0, wr0, wt0, b0, h0),
                                    (sp1, dg1, wr1, wt1, b1, h1),
                                    (sp2, dg2, wr2, wt2, b2, h2)):
        inv = 1.0 / jnp.maximum(jnp.sum(dg[...], axis=1, keepdims=True), 1.0)
        agg = (sp[0] + sp[1]) * inv
        out[...] = jnp.maximum(
            jnp.dot(agg, wr[...], preferred_element_type=jnp.float32)
            + jnp.dot(x, wt[...], preferred_element_type=jnp.float32)
            + bb[...], 0.0)


def _dense1_call(x, sp, dg, w):
    row = pl.BlockSpec((BROWS, D), lambda i: (i, 0))
    par = pl.BlockSpec((NC, BROWS, D), lambda i: (0, i, 0))
    degs = pl.BlockSpec((BROWS, NW), lambda i: (i, 0))
    mat = pl.BlockSpec((D, H), lambda i: (0, 0))
    vec = pl.BlockSpec((1, H), lambda i: (0, 0))
    return pl.pallas_call(
        _dense1_body,
        grid=(GRID,),
        in_specs=[row, par, par, par, degs, degs, degs] + [mat, mat, vec] * 3,
        out_specs=[row, row, row],
        out_shape=[jax.ShapeDtypeStruct((N, H), jnp.float32)] * 3,
    )(x, sp[0], sp[1], sp[2], dg[0], dg[1], dg[2],
      w[0][0], w[0][1], w[0][2],
      w[1][0], w[1][1], w[1][2],
      w[2][0], w[2][1], w[2][2])


def _dense2_body(h0r, h1r, h2r, tp0, tp1, tp2, dg0, dg1, dg2,
                 wr0, wt0, b0, wr1, wt1, b1, wr2, wt2, b2,
                 f10, f11, f12, f1b, w2p, b2p, out):
    g = []
    for hr, tp, dg, wr, wt, bb in ((h0r, tp0, dg0, wr0, wt0, b0),
                                   (h1r, tp1, dg1, wr1, wt1, b1),
                                   (h2r, tp2, dg2, wr2, wt2, b2)):
        inv = 1.0 / jnp.maximum(jnp.sum(dg[...], axis=1, keepdims=True), 1.0)
        agg = (tp[0] + tp[1]) * inv
        g.append(jnp.maximum(
            jnp.dot(agg, wr[...], preferred_element_type=jnp.float32)
            + jnp.dot(hr[...], wt[...], preferred_element_type=jnp.float32)
            + bb[...], 0.0))
    z = jnp.maximum(
        jnp.dot(g[0], f10[...], preferred_element_type=jnp.float32)
        + jnp.dot(g[1], f11[...], preferred_element_type=jnp.float32)
        + jnp.dot(g[2], f12[...], preferred_element_type=jnp.float32)
        + f1b[...], 0.0)
    logits = jnp.dot(z, w2p[...], preferred_element_type=jnp.float32) + b2p[...]
    m = jnp.max(logits, axis=1, keepdims=True)
    lse = m + jnp.log(jnp.sum(jnp.exp(logits - m), axis=1, keepdims=True))
    out[...] = (logits - lse)[:, :NCLS]


def _dense2_call(h, tp, dg, w, f1, f1b, w2p, b2p):
    row = pl.BlockSpec((BROWS, D), lambda i: (i, 0))
    par = pl.BlockSpec((NC, BROWS, D), lambda i: (0, i, 0))
    degs = pl.BlockSpec((BROWS, NW), lambda i: (i, 0))
    mat = pl.BlockSpec((D, H), lambda i: (0, 0))
    vec = pl.BlockSpec((1, H), lambda i: (0, 0))
    outs = pl.BlockSpec((BROWS, NCLS), lambda i: (i, 0))
    return pl.pallas_call(
        _dense2_body,
        grid=(GRID,),
        in_specs=[row, row, row, par, par, par, degs, degs, degs]
                 + [mat, mat, vec] * 3 + [mat, mat, mat, vec, mat, vec],
        out_specs=outs,
        out_shape=jax.ShapeDtypeStruct((N, NCLS), jnp.float32),
    )(h[0], h[1], h[2], tp[0], tp[1], tp[2], dg[0], dg[1], dg[2],
      w[0][0], w[0][1], w[0][2],
      w[1][0], w[1][1], w[1][2],
      w[2][0], w[2][1], w[2][2],
      f1[0], f1[1], f1[2], f1b, w2p, b2p)


def kernel(x, edge_index, edge_type,
           Wrel_0_0, Wroot_0_0, b_0_0, Wrel_0_1, Wroot_0_1, b_0_1,
           Wrel_1_0, Wroot_1_0, b_1_0, Wrel_1_1, Wroot_1_1, b_1_1,
           Wrel_2_0, Wroot_2_0, b_2_0, Wrel_2_1, Wroot_2_1, b_2_1,
           fc1_W, fc1_b, fc2_W, fc2_b):
    src = edge_index[0]
    dst = edge_index[1]

    bsrc, bdst, cnt = _bucket_call(src, dst, edge_type)
    bsrc = bsrc.reshape(NUM_REL, NW, NBLK, BLK)
    bdst = bdst.reshape(NUM_REL, NW, NBLK, BLK)
    cnt_r = [cnt[:, r * LANES:(r + 1) * LANES] for r in range(NUM_REL)]

    wrel = ((Wrel_0_0, Wrel_0_1), (Wrel_1_0, Wrel_1_1), (Wrel_2_0, Wrel_2_1))
    wroot = ((Wroot_0_0, Wroot_0_1), (Wroot_1_0, Wroot_1_1),
             (Wroot_2_0, Wroot_2_1))
    bias = ((b_0_0, b_0_1), (b_1_0, b_1_1), (b_2_0, b_2_1))

    sp, dg1, w1 = [], [], []
    for i, mp in enumerate(METAPATHS):
        r = mp[0]
        s, dp = _segsum_call(x, bsrc[r], bdst[r], cnt_r[r])
        sp.append(s)
        dg1.append(dp.T)
        w1.append((wrel[i][0][r], wroot[i][0], bias[i][0].reshape(1, H)))
    h = _dense1_call(x, sp, dg1, w1)

    tp, dg2, w2 = [], [], []
    for i, mp in enumerate(METAPATHS):
        r = mp[1]
        t, dp = _segsum_call(h[i], bsrc[r], bdst[r], cnt_r[r])
        tp.append(t)
        dg2.append(dp.T)
        w2.append((wrel[i][1][r], wroot[i][1], bias[i][1].reshape(1, H)))

    f1 = [fc1_W[i * H:(i + 1) * H] for i in range(3)]
    w2pad = jnp.zeros((H, H), jnp.float32).at[:, :NCLS].set(fc2_W)
    b2pad = jnp.full((1, H), -1e30, jnp.float32).at[0, :NCLS].set(fc2_b)

    return _dense2_call(h, tp, dg2, w2, f1, fc1_b.reshape(1, H), w2pad, b2pad)


# speedup vs baseline: 1.1473x; 1.1473x over previous
import jax
import jax.numpy as jnp
from jax import lax
from jax.experimental import pallas as pl
from jax.experimental.pallas import tpu as pltpu
from jax.experimental.pallas import tpu_sc as plsc

N = 10000
E = 320000
D = 128
H = 128
NUM_REL = 4
NCLS = 16
METAPATHS = ((0, 1), (2, 3), (1, 0))

NC = 2
NS = 16
NW = NC * NS
LANES = 16
CHUNK = E // NW
VPC = CHUNK // LANES
NPAD = 10240
TRASH = N
BLK = 128
NBLK = 81
CAP = NBLK * BLK
SLAB = NPAD // NS

BROWS = 1000
GRID = N // BROWS


def _mesh():
    return plsc.VectorSubcoreMesh(core_axis_name="c", subcore_axis_name="s",
                                  num_cores=NC, num_subcores=NS)


_SC_PARAMS = pltpu.CompilerParams(needs_layout_passes=False,
                                  use_tc_tiling_on_sc=False)


def _wid():
    return lax.axis_index("s") * NC + lax.axis_index("c")


def _bucket_body(src_hbm, dst_hbm, type_hbm, bsrc_hbm, bdst_hbm, cnt_hbm,
                 src_v, dst_v, type_v, bsrc_v, bdst_v, cnt_v):
    wid = _wid()
    base = wid * CHUNK
    pltpu.sync_copy(src_hbm.at[pl.ds(base, CHUNK)], src_v)
    pltpu.sync_copy(dst_hbm.at[pl.ds(base, CHUNK)], dst_v)
    pltpu.sync_copy(type_hbm.at[pl.ds(base, CHUNK)], type_v)

    trash = jnp.full((LANES,), TRASH, jnp.int32)
    zero = jnp.zeros((LANES,), jnp.int32)

    def prefill(i, _):
        for r in range(NUM_REL):
            bsrc_v[pl.ds(r * CAP + i * LANES, LANES)] = trash
            bdst_v[pl.ds(r * CAP + i * LANES, LANES)] = zero
        return 0

    lax.fori_loop(0, CAP // LANES, prefill, 0)

    one = jnp.ones((LANES,), jnp.int32)

    def step(i, offs):
        s = src_v[pl.ds(i * LANES, LANES)]
        d = dst_v[pl.ds(i * LANES, LANES)]
        t = type_v[pl.ds(i * LANES, LANES)]
        new = []
        for r in range(NUM_REL):
            m = t == r
            c = plsc.cumsum(jnp.where(m, one, zero))
            pos = offs[r] + c - 1 + r * CAP
            plsc.store_scatter(bsrc_v, [pos], s, mask=m)
            plsc.store_scatter(bdst_v, [pos], d, mask=m)
            new.append(offs[r] + plsc.all_reduce_population_count(m))
        return tuple(new)

    offs = lax.fori_loop(0, VPC, step,
                         tuple(jnp.zeros((LANES,), jnp.int32)
                               for _ in range(NUM_REL)))
    for r in range(NUM_REL):
        cnt_v[pl.ds(r * LANES, LANES)] = offs[r]
        pltpu.sync_copy(bsrc_v.at[pl.ds(r * CAP, CAP)], bsrc_hbm.at[r, wid])
        pltpu.sync_copy(bdst_v.at[pl.ds(r * CAP, CAP)], bdst_hbm.at[r, wid])
    pltpu.sync_copy(cnt_v, cnt_hbm.at[wid])


def _bucket_call(src, dst, etype):
    k = pl.kernel(
        _bucket_body,
        out_type=(
            jax.ShapeDtypeStruct((NUM_REL, NW, CAP), jnp.int32),
            jax.ShapeDtypeStruct((NUM_REL, NW, CAP), jnp.int32),
            jax.ShapeDtypeStruct((NW, NUM_REL * LANES), jnp.int32),
        ),
        mesh=_mesh(),
        compiler_params=_SC_PARAMS,
        scratch_types=[
            pltpu.VMEM((CHUNK,), jnp.int32),
            pltpu.VMEM((CHUNK,), jnp.int32),
            pltpu.VMEM((CHUNK,), jnp.int32),
            pltpu.VMEM((NUM_REL * CAP,), jnp.int32),
            pltpu.VMEM((NUM_REL * CAP,), jnp.int32),
            pltpu.VMEM((NUM_REL * LANES,), jnp.int32),
        ],
    )
    return k(src, dst, etype)


def _segsum_body(h_hbm, bsrc_hbm, bdst_hbm, cnt_hbm, out_hbm, degp_hbm,
                 idx_s2, idx_d2, rows_a, cnt_v, deg_v,
                 accum, sem_a):
    cid = lax.axis_index("c")
    sid = lax.axis_index("s")
    wid = sid * NC + cid

    zero = jnp.zeros((LANES,), jnp.float32)

    def zfill(i, _):
        for j in range(D // LANES):
            rows_a[i, pl.ds(j * LANES, LANES)] = zero
        return 0

    lax.fori_loop(0, BLK, zfill, 0)
    for k in range(SLAB // BLK):
        pltpu.sync_copy(rows_a, accum.at[pl.ds(sid * SLAB + k * BLK, BLK)])

    def dzfill(i, _):
        deg_v[pl.ds(i * LANES, LANES)] = zero
        return 0

    lax.fori_loop(0, NPAD // LANES, dzfill, 0)
    plsc.subcore_barrier()

    pltpu.sync_copy(cnt_hbm.at[wid], cnt_v)
    n = jnp.max(cnt_v[...])
    nblk = jnp.maximum((n + BLK - 1) >> 7, 1)
    nblk2 = (nblk + 1) >> 1

    ones = jnp.ones((LANES,), jnp.float32)

    def loop(i, _):
        pltpu.sync_copy(bsrc_hbm.at[wid, i], idx_s2)
        pltpu.sync_copy(bdst_hbm.at[wid, i], idx_d2)
        pltpu.async_copy(h_hbm.at[idx_d2], rows_a, sem_a)
        for j in range(BLK // LANES):
            s = idx_s2[pl.ds(j * LANES, LANES)]
            plsc.addupdate_scatter(deg_v, [s], ones)
        pltpu.make_async_copy(h_hbm.at[idx_d2], rows_a, sem_a).wait()
        pltpu.sync_copy(rows_a, accum.at[idx_s2], add=True)
        return 0

    lax.fori_loop(0, nblk, loop, 0)

    plsc.subcore_barrier()
    for k in range(SLAB // BLK):
        sl = pl.ds(sid * SLAB + k * BLK, BLK)
        pltpu.sync_copy(accum.at[sl], out_hbm.at[cid, sl])
    pltpu.sync_copy(deg_v, degp_hbm.at[wid])


def _segsum_call(h, bsrc_r, bdst_r, cnt_r):
    k = pl.kernel(
        _segsum_body,
        out_type=(
            jax.ShapeDtypeStruct((NC, NPAD, D), jnp.float32),
            jax.ShapeDtypeStruct((NW, NPAD), jnp.float32),
        ),
        mesh=_mesh(),
        compiler_params=_SC_PARAMS,
        scratch_types=[
            pltpu.VMEM((BLK,), jnp.int32),
            pltpu.VMEM((BLK,), jnp.int32),
            pltpu.VMEM((BLK, D), jnp.float32),
            pltpu.VMEM((LANES,), jnp.int32),
            pltpu.VMEM((NPAD,), jnp.float32),
            pltpu.VMEM_SHARED((NPAD, D), jnp.float32),
            pltpu.SemaphoreType.DMA,
        ],
    )
    return k(h, bsrc_r, bdst_r, cnt_r)


def _dense1_body(x_ref, sp0, sp1, sp2, dg0, dg1, dg2,
                 wr0, wt0, b0, wr1, wt1, b1, wr2, wt2, b2,
                 h0, h1, h2):
    x = x_ref[...]
    for sp, dg, wr, wt, bb, out in ((sp0, dg0, wr0, wt0, b0, h0),
                                    (sp1, dg1, wr1, wt1, b1, h1),
                                    (sp2, dg2, wr2, wt2, b2, h2)):
        inv = 1.0 / jnp.maximum(jnp.sum(dg[...], axis=1, keepdims=True), 1.0)
        agg = (sp[0] + sp[1]) * inv
        out[...] = jnp.maximum(
            jnp.dot(agg, wr[...], preferred_element_type=jnp.float32)
            + jnp.dot(x, wt[...], preferred_element_type=jnp.float32)
            + bb[...], 0.0)


def _dense1_call(x, sp, dg, w):
    row = pl.BlockSpec((BROWS, D), lambda i: (i, 0))
    par = pl.BlockSpec((NC, BROWS, D), lambda i: (0, i, 0))
    degs = pl.BlockSpec((BROWS, NW), lambda i: (i, 0))
    mat = pl.BlockSpec((D, H), lambda i: (0, 0))
    vec = pl.BlockSpec((1, H), lambda i: (0, 0))
    return pl.pallas_call(
        _dense1_body,
        grid=(GRID,),
        in_specs=[row, par, par, par, degs, degs, degs] + [mat, mat, vec] * 3,
        out_specs=[row, row, row],
        out_shape=[jax.ShapeDtypeStruct((N, H), jnp.float32)] * 3,
    )(x, sp[0], sp[1], sp[2], dg[0], dg[1], dg[2],
      w[0][0], w[0][1], w[0][2],
      w[1][0], w[1][1], w[1][2],
      w[2][0], w[2][1], w[2][2])


def _dense2_body(h0r, h1r, h2r, tp0, tp1, tp2, dg0, dg1, dg2,
                 wr0, wt0, b0, wr1, wt1, b1, wr2, wt2, b2,
                 f10, f11, f12, f1b, w2p, b2p, out):
    g = []
    for hr, tp, dg, wr, wt, bb in ((h0r, tp0, dg0, wr0, wt0, b0),
                                   (h1r, tp1, dg1, wr1, wt1, b1),
                                   (h2r, tp2, dg2, wr2, wt2, b2)):
        inv = 1.0 / jnp.maximum(jnp.sum(dg[...], axis=1, keepdims=True), 1.0)
        agg = (tp[0] + tp[1]) * inv
        g.append(jnp.maximum(
            jnp.dot(agg, wr[...], preferred_element_type=jnp.float32)
            + jnp.dot(hr[...], wt[...], preferred_element_type=jnp.float32)
            + bb[...], 0.0))
    z = jnp.maximum(
        jnp.dot(g[0], f10[...], preferred_element_type=jnp.float32)
        + jnp.dot(g[1], f11[...], preferred_element_type=jnp.float32)
        + jnp.dot(g[2], f12[...], preferred_element_type=jnp.float32)
        + f1b[...], 0.0)
    logits = jnp.dot(z, w2p[...], preferred_element_type=jnp.float32) + b2p[...]
    m = jnp.max(logits, axis=1, keepdims=True)
    lse = m + jnp.log(jnp.sum(jnp.exp(logits - m), axis=1, keepdims=True))
    out[...] = (logits - lse)[:, :NCLS]


def _dense2_call(h, tp, dg, w, f1, f1b, w2p, b2p):
    row = pl.BlockSpec((BROWS, D), lambda i: (i, 0))
    par = pl.BlockSpec((NC, BROWS, D), lambda i: (0, i, 0))
    degs = pl.BlockSpec((BROWS, NW), lambda i: (i, 0))
    mat = pl.BlockSpec((D, H), lambda i: (0, 0))
    vec = pl.BlockSpec((1, H), lambda i: (0, 0))
    outs = pl.BlockSpec((BROWS, NCLS), lambda i: (i, 0))
    return pl.pallas_call(
        _dense2_body,
        grid=(GRID,),
        in_specs=[row, row, row, par, par, par, degs, degs, degs]
                 + [mat, mat, vec] * 3 + [mat, mat, mat, vec, mat, vec],
        out_specs=outs,
        out_shape=jax.ShapeDtypeStruct((N, NCLS), jnp.float32),
    )(h[0], h[1], h[2], tp[0], tp[1], tp[2], dg[0], dg[1], dg[2],
      w[0][0], w[0][1], w[0][2],
      w[1][0], w[1][1], w[1][2],
      w[2][0], w[2][1], w[2][2],
      f1[0], f1[1], f1[2], f1b, w2p, b2p)


def kernel(x, edge_index, edge_type,
           Wrel_0_0, Wroot_0_0, b_0_0, Wrel_0_1, Wroot_0_1, b_0_1,
           Wrel_1_0, Wroot_1_0, b_1_0, Wrel_1_1, Wroot_1_1, b_1_1,
           Wrel_2_0, Wroot_2_0, b_2_0, Wrel_2_1, Wroot_2_1, b_2_1,
           fc1_W, fc1_b, fc2_W, fc2_b):
    src = edge_index[0]
    dst = edge_index[1]

    bsrc, bdst, cnt = _bucket_call(src, dst, edge_type)
    bsrc = bsrc.reshape(NUM_REL, NW, NBLK, BLK)
    bdst = bdst.reshape(NUM_REL, NW, NBLK, BLK)
    cnt_r = [cnt[:, r * LANES:(r + 1) * LANES] for r in range(NUM_REL)]

    wrel = ((Wrel_0_0, Wrel_0_1), (Wrel_1_0, Wrel_1_1), (Wrel_2_0, Wrel_2_1))
    wroot = ((Wroot_0_0, Wroot_0_1), (Wroot_1_0, Wroot_1_1),
             (Wroot_2_0, Wroot_2_1))
    bias = ((b_0_0, b_0_1), (b_1_0, b_1_1), (b_2_0, b_2_1))

    sp, dg1, w1 = [], [], []
    for i, mp in enumerate(METAPATHS):
        r = mp[0]
        s, dp = _segsum_call(x, bsrc[r], bdst[r], cnt_r[r])
        sp.append(s)
        dg1.append(dp.T)
        w1.append((wrel[i][0][r], wroot[i][0], bias[i][0].reshape(1, H)))
    h = _dense1_call(x, sp, dg1, w1)

    tp, dg2, w2 = [], [], []
    for i, mp in enumerate(METAPATHS):
        r = mp[1]
        t, dp = _segsum_call(h[i], bsrc[r], bdst[r], cnt_r[r])
        tp.append(t)
        dg2.append(dp.T)
        w2.append((wrel[i][1][r], wroot[i][1], bias[i][1].reshape(1, H)))

    f1 = [fc1_W[i * H:(i + 1) * H] for i in range(3)]
    w2pad = jnp.zeros((H, H), jnp.float32).at[:, :NCLS].set(fc2_W)
    b2pad = jnp.full((1, H), -1e30, jnp.float32).at[0, :NCLS].set(fc2_b)

    return _dense2_call(h, tp, dg2, w2, f1, fc1_b.reshape(1, H), w2pad, b2pad)


# speedup vs baseline: 1.2248x; 1.0675x over previous
import jax
import jax.numpy as jnp
from jax import lax
from jax.experimental import pallas as pl
from jax.experimental.pallas import tpu as pltpu
from jax.experimental.pallas import tpu_sc as plsc

N = 10000
E = 320000
D = 128
H = 128
NUM_REL = 4
NCLS = 16
METAPATHS = ((0, 1), (2, 3), (1, 0))

NC = 2
NS = 16
NW = NC * NS
LANES = 16
CHUNK = E // NW
VPC = CHUNK // LANES
NPAD = 10240
TRASH = N
BLK = 128
NBLK = 81
CAP = NBLK * BLK
SLAB = NPAD // NS

BROWS = 1000
GRID = N // BROWS


def _mesh():
    return plsc.VectorSubcoreMesh(core_axis_name="c", subcore_axis_name="s",
                                  num_cores=NC, num_subcores=NS)


_SC_PARAMS = pltpu.CompilerParams(needs_layout_passes=False,
                                  use_tc_tiling_on_sc=False)


def _wid():
    return lax.axis_index("s") * NC + lax.axis_index("c")


def _bucket_body(src_hbm, dst_hbm, type_hbm, bsrc_hbm, bdst_hbm, cnt_hbm,
                 src_v, dst_v, type_v, bsrc_v, bdst_v, cnt_v):
    wid = _wid()
    base = wid * CHUNK
    pltpu.sync_copy(src_hbm.at[pl.ds(base, CHUNK)], src_v)
    pltpu.sync_copy(dst_hbm.at[pl.ds(base, CHUNK)], dst_v)
    pltpu.sync_copy(type_hbm.at[pl.ds(base, CHUNK)], type_v)

    trash = TRASH + jnp.arange(LANES, dtype=jnp.int32) * 8
    zero = jnp.zeros((LANES,), jnp.int32)

    def prefill(i, _):
        for r in range(NUM_REL):
            bsrc_v[pl.ds(r * CAP + i * LANES, LANES)] = trash
            bdst_v[pl.ds(r * CAP + i * LANES, LANES)] = zero
        return 0

    lax.fori_loop(0, CAP // LANES, prefill, 0)

    one = jnp.ones((LANES,), jnp.int32)

    def step(i, offs):
        s = src_v[pl.ds(i * LANES, LANES)]
        d = dst_v[pl.ds(i * LANES, LANES)]
        t = type_v[pl.ds(i * LANES, LANES)]
        sh = t * 8
        cp = plsc.cumsum(jnp.left_shift(one, sh))
        rank = jnp.right_shift(cp, sh) & 255
        total = jnp.max(cp)
        offv = jnp.where(t == 0, offs[0],
                         jnp.where(t == 1, offs[1],
                                   jnp.where(t == 2, offs[2], offs[3])))
        pos = t * CAP + offv + rank - 1
        plsc.store_scatter(bsrc_v, [pos], s)
        plsc.store_scatter(bdst_v, [pos], d)
        return (offs[0] + (total & 255),
                offs[1] + (jnp.right_shift(total, 8) & 255),
                offs[2] + (jnp.right_shift(total, 16) & 255),
                offs[3] + jnp.right_shift(total, 24))

    offs = lax.fori_loop(0, VPC, step,
                         tuple(jnp.int32(0) for _ in range(NUM_REL)))
    for r in range(NUM_REL):
        cnt_v[pl.ds(r * LANES, LANES)] = zero + offs[r]
        pltpu.sync_copy(bsrc_v.at[pl.ds(r * CAP, CAP)], bsrc_hbm.at[r, wid])
        pltpu.sync_copy(bdst_v.at[pl.ds(r * CAP, CAP)], bdst_hbm.at[r, wid])
    pltpu.sync_copy(cnt_v, cnt_hbm.at[wid])


def _bucket_call(src, dst, etype):
    k = pl.kernel(
        _bucket_body,
        out_type=(
            jax.ShapeDtypeStruct((NUM_REL, NW, CAP), jnp.int32),
            jax.ShapeDtypeStruct((NUM_REL, NW, CAP), jnp.int32),
            jax.ShapeDtypeStruct((NW, NUM_REL * LANES), jnp.int32),
        ),
        mesh=_mesh(),
        compiler_params=_SC_PARAMS,
        scratch_types=[
            pltpu.VMEM((CHUNK,), jnp.int32),
            pltpu.VMEM((CHUNK,), jnp.int32),
            pltpu.VMEM((CHUNK,), jnp.int32),
            pltpu.VMEM((NUM_REL * CAP,), jnp.int32),
            pltpu.VMEM((NUM_REL * CAP,), jnp.int32),
            pltpu.VMEM((NUM_REL * LANES,), jnp.int32),
        ],
    )
    return k(src, dst, etype)


def _segsum_body(h_hbm, bsrc_hbm, bdst_hbm, cnt_hbm, out_hbm, degp_hbm,
                 idx_s2, idx_d2, rows_a, cnt_v, deg_v,
                 accum, sem_a):
    cid = lax.axis_index("c")
    sid = lax.axis_index("s")
    wid = sid * NC + cid

    zero = jnp.zeros((LANES,), jnp.float32)

    def zfill(i, _):
        for j in range(D // LANES):
            rows_a[i, pl.ds(j * LANES, LANES)] = zero
        return 0

    lax.fori_loop(0, BLK, zfill, 0)
    for k in range(SLAB // BLK):
        pltpu.sync_copy(rows_a, accum.at[pl.ds(sid * SLAB + k * BLK, BLK)])

    def dzfill(i, _):
        deg_v[pl.ds(i * LANES, LANES)] = zero
        return 0

    lax.fori_loop(0, NPAD // LANES, dzfill, 0)
    plsc.subcore_barrier()

    pltpu.sync_copy(cnt_hbm.at[wid], cnt_v)
    n = jnp.max(cnt_v[...])
    nblk = jnp.maximum((n + BLK - 1) >> 7, 1)
    ngrp = nblk >> 2

    ones = jnp.ones((LANES,), jnp.float32)

    def one_block(k):
        pltpu.async_copy(h_hbm.at[idx_d2.at[k]], rows_a, sem_a)
        for j in range(BLK // LANES):
            s = idx_s2[k, pl.ds(j * LANES, LANES)]
            plsc.addupdate_scatter(deg_v, [s], ones)
        pltpu.make_async_copy(h_hbm.at[idx_d2.at[k]], rows_a, sem_a).wait()
        pltpu.sync_copy(rows_a, accum.at[idx_s2.at[k]], add=True)

    def gloop(g, _):
        pltpu.sync_copy(bsrc_hbm.at[wid, pl.ds(g * 4, 4)], idx_s2)
        pltpu.sync_copy(bdst_hbm.at[wid, pl.ds(g * 4, 4)], idx_d2)
        for k in range(4):
            one_block(k)
        return 0

    lax.fori_loop(0, ngrp, gloop, 0)

    def tloop(b, _):
        pltpu.sync_copy(bsrc_hbm.at[wid, b], idx_s2.at[0])
        pltpu.sync_copy(bdst_hbm.at[wid, b], idx_d2.at[0])
        one_block(0)
        return 0

    lax.fori_loop(ngrp * 4, nblk, tloop, 0)

    plsc.subcore_barrier()
    for k in range(SLAB // BLK):
        sl = pl.ds(sid * SLAB + k * BLK, BLK)
        pltpu.sync_copy(accum.at[sl], out_hbm.at[cid, sl])
    pltpu.sync_copy(deg_v, degp_hbm.at[wid])


def _segsum_call(h, bsrc_r, bdst_r, cnt_r):
    k = pl.kernel(
        _segsum_body,
        out_type=(
            jax.ShapeDtypeStruct((NC, NPAD, D), jnp.float32),
            jax.ShapeDtypeStruct((NW, NPAD), jnp.float32),
        ),
        mesh=_mesh(),
        compiler_params=_SC_PARAMS,
        scratch_types=[
            pltpu.VMEM((4, BLK), jnp.int32),
            pltpu.VMEM((4, BLK), jnp.int32),
            pltpu.VMEM((BLK, D), jnp.float32),
            pltpu.VMEM((LANES,), jnp.int32),
            pltpu.VMEM((NPAD,), jnp.float32),
            pltpu.VMEM_SHARED((NPAD, D), jnp.float32),
            pltpu.SemaphoreType.DMA,
        ],
    )
    return k(h, bsrc_r, bdst_r, cnt_r)


def _dense1_body(x_ref, sp0, sp1, sp2, dg0, dg1, dg2,
                 wr0, wt0, b0, wr1, wt1, b1, wr2, wt2, b2,
                 h0, h1, h2):
    x = x_ref[...]
    for sp, dg, wr, wt, bb, out in ((sp0, dg0, wr0, wt0, b0, h0),
                                    (sp1, dg1, wr1, wt1, b1, h1),
                                    (sp2, dg2, wr2, wt2, b2, h2)):
        inv = 1.0 / jnp.maximum(jnp.sum(dg[...], axis=1, keepdims=True), 1.0)
        agg = (sp[0] + sp[1]) * inv
        out[...] = jnp.maximum(
            jnp.dot(agg, wr[...], preferred_element_type=jnp.float32)
            + jnp.dot(x, wt[...], preferred_element_type=jnp.float32)
            + bb[...], 0.0)


def _dense1_call(x, sp, dg, w):
    row = pl.BlockSpec((BROWS, D), lambda i: (i, 0))
    par = pl.BlockSpec((NC, BROWS, D), lambda i: (0, i, 0))
    degs = pl.BlockSpec((BROWS, NW), lambda i: (i, 0))
    mat = pl.BlockSpec((D, H), lambda i: (0, 0))
    vec = pl.BlockSpec((1, H), lambda i: (0, 0))
    return pl.pallas_call(
        _dense1_body,
        grid=(GRID,),
        in_specs=[row, par, par, par, degs, degs, degs] + [mat, mat, vec] * 3,
        out_specs=[row, row, row],
        out_shape=[jax.ShapeDtypeStruct((N, H), jnp.float32)] * 3,
    )(x, sp[0], sp[1], sp[2], dg[0], dg[1], dg[2],
      w[0][0], w[0][1], w[0][2],
      w[1][0], w[1][1], w[1][2],
      w[2][0], w[2][1], w[2][2])


def _dense2_body(h0r, h1r, h2r, tp0, tp1, tp2, dg0, dg1, dg2,
                 wr0, wt0, b0, wr1, wt1, b1, wr2, wt2, b2,
                 f10, f11, f12, f1b, w2p, b2p, out):
    g = []
    for hr, tp, dg, wr, wt, bb in ((h0r, tp0, dg0, wr0, wt0, b0),
                                   (h1r, tp1, dg1, wr1, wt1, b1),
                                   (h2r, tp2, dg2, wr2, wt2, b2)):
        inv = 1.0 / jnp.maximum(jnp.sum(dg[...], axis=1, keepdims=True), 1.0)
        agg = (tp[0] + tp[1]) * inv
        g.append(jnp.maximum(
            jnp.dot(agg, wr[...], preferred_element_type=jnp.float32)
            + jnp.dot(hr[...], wt[...], preferred_element_type=jnp.float32)
            + bb[...], 0.0))
    z = jnp.maximum(
        jnp.dot(g[0], f10[...], preferred_element_type=jnp.float32)
        + jnp.dot(g[1], f11[...], preferred_element_type=jnp.float32)
        + jnp.dot(g[2], f12[...], preferred_element_type=jnp.float32)
        + f1b[...], 0.0)
    logits = jnp.dot(z, w2p[...], preferred_element_type=jnp.float32) + b2p[...]
    m = jnp.max(logits, axis=1, keepdims=True)
    lse = m + jnp.log(jnp.sum(jnp.exp(logits - m), axis=1, keepdims=True))
    out[...] = (logits - lse)[:, :NCLS]


def _dense2_call(h, tp, dg, w, f1, f1b, w2p, b2p):
    row = pl.BlockSpec((BROWS, D), lambda i: (i, 0))
    par = pl.BlockSpec((NC, BROWS, D), lambda i: (0, i, 0))
    degs = pl.BlockSpec((BROWS, NW), lambda i: (i, 0))
    mat = pl.BlockSpec((D, H), lambda i: (0, 0))
    vec = pl.BlockSpec((1, H), lambda i: (0, 0))
    outs = pl.BlockSpec((BROWS, NCLS), lambda i: (i, 0))
    return pl.pallas_call(
        _dense2_body,
        grid=(GRID,),
        in_specs=[row, row, row, par, par, par, degs, degs, degs]
                 + [mat, mat, vec] * 3 + [mat, mat, mat, vec, mat, vec],
        out_specs=outs,
        out_shape=jax.ShapeDtypeStruct((N, NCLS), jnp.float32),
    )(h[0], h[1], h[2], tp[0], tp[1], tp[2], dg[0], dg[1], dg[2],
      w[0][0], w[0][1], w[0][2],
      w[1][0], w[1][1], w[1][2],
      w[2][0], w[2][1], w[2][2],
      f1[0], f1[1], f1[2], f1b, w2p, b2p)


def kernel(x, edge_index, edge_type,
           Wrel_0_0, Wroot_0_0, b_0_0, Wrel_0_1, Wroot_0_1, b_0_1,
           Wrel_1_0, Wroot_1_0, b_1_0, Wrel_1_1, Wroot_1_1, b_1_1,
           Wrel_2_0, Wroot_2_0, b_2_0, Wrel_2_1, Wroot_2_1, b_2_1,
           fc1_W, fc1_b, fc2_W, fc2_b):
    src = edge_index[0]
    dst = edge_index[1]

    bsrc, bdst, cnt = _bucket_call(src, dst, edge_type)
    bsrc = bsrc.reshape(NUM_REL, NW, NBLK, BLK)
    bdst = bdst.reshape(NUM_REL, NW, NBLK, BLK)
    cnt_r = [cnt[:, r * LANES:(r + 1) * LANES] for r in range(NUM_REL)]

    wrel = ((Wrel_0_0, Wrel_0_1), (Wrel_1_0, Wrel_1_1), (Wrel_2_0, Wrel_2_1))
    wroot = ((Wroot_0_0, Wroot_0_1), (Wroot_1_0, Wroot_1_1),
             (Wroot_2_0, Wroot_2_1))
    bias = ((b_0_0, b_0_1), (b_1_0, b_1_1), (b_2_0, b_2_1))

    sp, dg1, w1 = [], [], []
    for i, mp in enumerate(METAPATHS):
        r = mp[0]
        s, dp = _segsum_call(x, bsrc[r], bdst[r], cnt_r[r])
        sp.append(s)
        dg1.append(dp.T)
        w1.append((wrel[i][0][r], wroot[i][0], bias[i][0].reshape(1, H)))
    h = _dense1_call(x, sp, dg1, w1)

    tp, dg2, w2 = [], [], []
    for i, mp in enumerate(METAPATHS):
        r = mp[1]
        t, dp = _segsum_call(h[i], bsrc[r], bdst[r], cnt_r[r])
        tp.append(t)
        dg2.append(dp.T)
        w2.append((wrel[i][1][r], wroot[i][1], bias[i][1].reshape(1, H)))

    f1 = [fc1_W[i * H:(i + 1) * H] for i in range(3)]
    w2pad = jnp.zeros((H, H), jnp.float32).at[:, :NCLS].set(fc2_W)
    b2pad = jnp.full((1, H), -1e30, jnp.float32).at[0, :NCLS].set(fc2_b)

    return _dense2_call(h, tp, dg2, w2, f1, fc1_b.reshape(1, H), w2pad, b2pad)


# speedup vs baseline: 1.3398x; 1.0939x over previous
import jax
import jax.numpy as jnp
from jax import lax
from jax.experimental import pallas as pl
from jax.experimental.pallas import tpu as pltpu
from jax.experimental.pallas import tpu_sc as plsc

N = 10000
E = 320000
D = 128
H = 128
NUM_REL = 4
NCLS = 16
METAPATHS = ((0, 1), (2, 3), (1, 0))

NC = 2
NS = 16
NW = NC * NS
LANES = 16
CHUNK = E // NW
VPC = CHUNK // LANES
NPAD = 10240
TRASH = N
BLK = 128
NBLK = 81
CAP = NBLK * BLK
SLAB = NPAD // NS

BROWS = 1000
GRID = N // BROWS


def _mesh():
    return plsc.VectorSubcoreMesh(core_axis_name="c", subcore_axis_name="s",
                                  num_cores=NC, num_subcores=NS)


_SC_PARAMS = pltpu.CompilerParams(needs_layout_passes=False,
                                  use_tc_tiling_on_sc=False)


def _wid():
    return lax.axis_index("s") * NC + lax.axis_index("c")


def _bucket_body(src_hbm, dst_hbm, type_hbm, bsrc_hbm, bdst_hbm, cnt_hbm,
                 src_v, dst_v, type_v, bsrc_v, bdst_v, cnt_v):
    wid = _wid()
    base = wid * CHUNK
    pltpu.sync_copy(src_hbm.at[pl.ds(base, CHUNK)], src_v)
    pltpu.sync_copy(dst_hbm.at[pl.ds(base, CHUNK)], dst_v)
    pltpu.sync_copy(type_hbm.at[pl.ds(base, CHUNK)], type_v)

    trash = TRASH + jnp.arange(LANES, dtype=jnp.int32) * 8
    zero = jnp.zeros((LANES,), jnp.int32)

    def prefill(i, _):
        for r in range(NUM_REL):
            bsrc_v[pl.ds(r * CAP + i * LANES, LANES)] = trash
            bdst_v[pl.ds(r * CAP + i * LANES, LANES)] = zero
        return 0

    lax.fori_loop(0, CAP // LANES, prefill, 0)

    one = jnp.ones((LANES,), jnp.int32)

    def step(i, offs):
        s = src_v[pl.ds(i * LANES, LANES)]
        d = dst_v[pl.ds(i * LANES, LANES)]
        t = type_v[pl.ds(i * LANES, LANES)]
        sh = t * 8
        cp = plsc.cumsum(jnp.left_shift(one, sh))
        rank = jnp.right_shift(cp, sh) & 255
        total = jnp.max(cp)
        offv = jnp.where(t == 0, offs[0],
                         jnp.where(t == 1, offs[1],
                                   jnp.where(t == 2, offs[2], offs[3])))
        pos = t * CAP + offv + rank - 1
        plsc.store_scatter(bsrc_v, [pos], s)
        plsc.store_scatter(bdst_v, [pos], d)
        return (offs[0] + (total & 255),
                offs[1] + (jnp.right_shift(total, 8) & 255),
                offs[2] + (jnp.right_shift(total, 16) & 255),
                offs[3] + jnp.right_shift(total, 24))

    offs = lax.fori_loop(0, VPC, step,
                         tuple(jnp.int32(0) for _ in range(NUM_REL)))
    for r in range(NUM_REL):
        cnt_v[pl.ds(r * LANES, LANES)] = zero + offs[r]
        pltpu.sync_copy(bsrc_v.at[pl.ds(r * CAP, CAP)], bsrc_hbm.at[r, wid])
        pltpu.sync_copy(bdst_v.at[pl.ds(r * CAP, CAP)], bdst_hbm.at[r, wid])
    pltpu.sync_copy(cnt_v, cnt_hbm.at[wid])


def _bucket_call(src, dst, etype):
    k = pl.kernel(
        _bucket_body,
        out_type=(
            jax.ShapeDtypeStruct((NUM_REL, NW, CAP), jnp.int32),
            jax.ShapeDtypeStruct((NUM_REL, NW, CAP), jnp.int32),
            jax.ShapeDtypeStruct((NW, NUM_REL * LANES), jnp.int32),
        ),
        mesh=_mesh(),
        compiler_params=_SC_PARAMS,
        scratch_types=[
            pltpu.VMEM((CHUNK,), jnp.int32),
            pltpu.VMEM((CHUNK,), jnp.int32),
            pltpu.VMEM((CHUNK,), jnp.int32),
            pltpu.VMEM((NUM_REL * CAP,), jnp.int32),
            pltpu.VMEM((NUM_REL * CAP,), jnp.int32),
            pltpu.VMEM((NUM_REL * LANES,), jnp.int32),
        ],
    )
    return k(src, dst, etype)


def _make_segsum3(rels, nh):
    def body(*args):
        hs = args[:nh]
        bsrc_hbm, bdst_hbm, cnt_hbm = args[nh:nh + 3]
        outs = args[nh + 3:nh + 6]
        degps = args[nh + 6:nh + 9]
        (idx_s2, idx_d2, rows_a, rows_b, cnt_v, deg_v, accum,
         sem_a, sem_b) = args[nh + 9:]

        cid = lax.axis_index("c")
        sid = lax.axis_index("s")
        wid = sid * NC + cid

        zero = jnp.zeros((LANES,), jnp.float32)
        ones = jnp.ones((LANES,), jnp.float32)
        pltpu.sync_copy(cnt_hbm.at[wid], cnt_v)

        for p, r in enumerate(rels):
            h_hbm = hs[p] if nh == 3 else hs[0]
            out_hbm = outs[p]
            degp_hbm = degps[p]

            def zfill(i, _):
                for j in range(D // LANES):
                    rows_a[i, pl.ds(j * LANES, LANES)] = zero
                return 0

            lax.fori_loop(0, BLK, zfill, 0)
            for k in range(SLAB // BLK):
                pltpu.sync_copy(rows_a,
                                accum.at[pl.ds(sid * SLAB + k * BLK, BLK)])

            def dzfill(i, _):
                deg_v[pl.ds(i * LANES, LANES)] = zero
                return 0

            lax.fori_loop(0, NPAD // LANES, dzfill, 0)
            plsc.subcore_barrier()

            n = jnp.max(cnt_v[pl.ds(r * LANES, LANES)])
            nblk = jnp.maximum((n + BLK - 1) >> 7, 1)
            ngrp = nblk >> 2

            bufs = ((rows_a, sem_a), (rows_b, sem_b))

            def start(k):
                rows, sem = bufs[k % 2]
                pltpu.async_copy(h_hbm.at[idx_d2.at[k]], rows, sem)

            def gloop(g, _):
                pltpu.sync_copy(bsrc_hbm.at[r, wid, pl.ds(g * 4, 4)], idx_s2)
                pltpu.sync_copy(bdst_hbm.at[r, wid, pl.ds(g * 4, 4)], idx_d2)
                start(0)
                for k in range(4):
                    rows, sem = bufs[k % 2]
                    pltpu.make_async_copy(h_hbm.at[idx_d2.at[k]], rows,
                                          sem).wait()
                    if k < 3:
                        start(k + 1)
                    for j in range(BLK // LANES):
                        s = idx_s2[k, pl.ds(j * LANES, LANES)]
                        plsc.addupdate_scatter(deg_v, [s], ones)
                    pltpu.sync_copy(rows, accum.at[idx_s2.at[k]], add=True)
                return 0

            lax.fori_loop(0, ngrp, gloop, 0)

            def tloop(b, _):
                pltpu.sync_copy(bsrc_hbm.at[r, wid, b], idx_s2.at[0])
                pltpu.sync_copy(bdst_hbm.at[r, wid, b], idx_d2.at[0])
                start(0)
                for j in range(BLK // LANES):
                    s = idx_s2[0, pl.ds(j * LANES, LANES)]
                    plsc.addupdate_scatter(deg_v, [s], ones)
                pltpu.make_async_copy(h_hbm.at[idx_d2.at[0]], rows_a,
                                      sem_a).wait()
                pltpu.sync_copy(rows_a, accum.at[idx_s2.at[0]], add=True)
                return 0

            lax.fori_loop(ngrp * 4, nblk, tloop, 0)

            plsc.subcore_barrier()
            for k in range(SLAB // BLK):
                sl = pl.ds(sid * SLAB + k * BLK, BLK)
                pltpu.sync_copy(accum.at[sl], out_hbm.at[cid, sl])
            pltpu.sync_copy(deg_v, degp_hbm.at[wid])

    return pl.kernel(
        body,
        out_type=(
            [jax.ShapeDtypeStruct((NC, NPAD, D), jnp.float32)] * 3
            + [jax.ShapeDtypeStruct((NW, NPAD), jnp.float32)] * 3
        ),
        mesh=_mesh(),
        compiler_params=_SC_PARAMS,
        scratch_types=[
            pltpu.VMEM((4, BLK), jnp.int32),
            pltpu.VMEM((4, BLK), jnp.int32),
            pltpu.VMEM((BLK, D), jnp.float32),
            pltpu.VMEM((BLK, D), jnp.float32),
            pltpu.VMEM((NUM_REL * LANES,), jnp.int32),
            pltpu.VMEM((NPAD,), jnp.float32),
            pltpu.VMEM_SHARED((NPAD, D), jnp.float32),
            pltpu.SemaphoreType.DMA,
            pltpu.SemaphoreType.DMA,
        ],
    )


def _dense1_body(x_ref, sp0, sp1, sp2, dg0, dg1, dg2,
                 wr0, wt0, b0, wr1, wt1, b1, wr2, wt2, b2,
                 h0, h1, h2):
    x = x_ref[...]
    for sp, dg, wr, wt, bb, out in ((sp0, dg0, wr0, wt0, b0, h0),
                                    (sp1, dg1, wr1, wt1, b1, h1),
                                    (sp2, dg2, wr2, wt2, b2, h2)):
        inv = 1.0 / jnp.maximum(jnp.sum(dg[...], axis=1, keepdims=True), 1.0)
        agg = (sp[0] + sp[1]) * inv
        out[...] = jnp.maximum(
            jnp.dot(agg, wr[...], preferred_element_type=jnp.float32)
            + jnp.dot(x, wt[...], preferred_element_type=jnp.float32)
            + bb[...], 0.0)


def _dense1_call(x, sp, dg, w):
    row = pl.BlockSpec((BROWS, D), lambda i: (i, 0))
    par = pl.BlockSpec((NC, BROWS, D), lambda i: (0, i, 0))
    degs = pl.BlockSpec((BROWS, NW), lambda i: (i, 0))
    mat = pl.BlockSpec((D, H), lambda i: (0, 0))
    vec = pl.BlockSpec((1, H), lambda i: (0, 0))
    return pl.pallas_call(
        _dense1_body,
        grid=(GRID,),
        in_specs=[row, par, par, par, degs, degs, degs] + [mat, mat, vec] * 3,
        out_specs=[row, row, row],
        out_shape=[jax.ShapeDtypeStruct((N, H), jnp.float32)] * 3,
    )(x, sp[0], sp[1], sp[2], dg[0], dg[1], dg[2],
      w[0][0], w[0][1], w[0][2],
      w[1][0], w[1][1], w[1][2],
      w[2][0], w[2][1], w[2][2])


def _dense2_body(h0r, h1r, h2r, tp0, tp1, tp2, dg0, dg1, dg2,
                 wr0, wt0, b0, wr1, wt1, b1, wr2, wt2, b2,
                 f10, f11, f12, f1b, w2p, b2p, out):
    g = []
    for hr, tp, dg, wr, wt, bb in ((h0r, tp0, dg0, wr0, wt0, b0),
                                   (h1r, tp1, dg1, wr1, wt1, b1),
                                   (h2r, tp2, dg2, wr2, wt2, b2)):
        inv = 1.0 / jnp.maximum(jnp.sum(dg[...], axis=1, keepdims=True), 1.0)
        agg = (tp[0] + tp[1]) * inv
        g.append(jnp.maximum(
            jnp.dot(agg, wr[...], preferred_element_type=jnp.float32)
            + jnp.dot(hr[...], wt[...], preferred_element_type=jnp.float32)
            + bb[...], 0.0))
    z = jnp.maximum(
        jnp.dot(g[0], f10[...], preferred_element_type=jnp.float32)
        + jnp.dot(g[1], f11[...], preferred_element_type=jnp.float32)
        + jnp.dot(g[2], f12[...], preferred_element_type=jnp.float32)
        + f1b[...], 0.0)
    logits = jnp.dot(z, w2p[...], preferred_element_type=jnp.float32) + b2p[...]
    m = jnp.max(logits, axis=1, keepdims=True)
    lse = m + jnp.log(jnp.sum(jnp.exp(logits - m), axis=1, keepdims=True))
    out[...] = (logits - lse)[:, :NCLS]


def _dense2_call(h, tp, dg, w, f1, f1b, w2p, b2p):
    row = pl.BlockSpec((BROWS, D), lambda i: (i, 0))
    par = pl.BlockSpec((NC, BROWS, D), lambda i: (0, i, 0))
    degs = pl.BlockSpec((BROWS, NW), lambda i: (i, 0))
    mat = pl.BlockSpec((D, H), lambda i: (0, 0))
    vec = pl.BlockSpec((1, H), lambda i: (0, 0))
    outs = pl.BlockSpec((BROWS, NCLS), lambda i: (i, 0))
    return pl.pallas_call(
        _dense2_body,
        grid=(GRID,),
        in_specs=[row, row, row, par, par, par, degs, degs, degs]
                 + [mat, mat, vec] * 3 + [mat, mat, mat, vec, mat, vec],
        out_specs=outs,
        out_shape=jax.ShapeDtypeStruct((N, NCLS), jnp.float32),
    )(h[0], h[1], h[2], tp[0], tp[1], tp[2], dg[0], dg[1], dg[2],
      w[0][0], w[0][1], w[0][2],
      w[1][0], w[1][1], w[1][2],
      w[2][0], w[2][1], w[2][2],
      f1[0], f1[1], f1[2], f1b, w2p, b2p)


def kernel(x, edge_index, edge_type,
           Wrel_0_0, Wroot_0_0, b_0_0, Wrel_0_1, Wroot_0_1, b_0_1,
           Wrel_1_0, Wroot_1_0, b_1_0, Wrel_1_1, Wroot_1_1, b_1_1,
           Wrel_2_0, Wroot_2_0, b_2_0, Wrel_2_1, Wroot_2_1, b_2_1,
           fc1_W, fc1_b, fc2_W, fc2_b):
    src = edge_index[0]
    dst = edge_index[1]

    bsrc, bdst, cnt = _bucket_call(src, dst, edge_type)
    bsrc = bsrc.reshape(NUM_REL, NW, NBLK, BLK)
    bdst = bdst.reshape(NUM_REL, NW, NBLK, BLK)
    cnt_r = [cnt[:, r * LANES:(r + 1) * LANES] for r in range(NUM_REL)]

    wrel = ((Wrel_0_0, Wrel_0_1), (Wrel_1_0, Wrel_1_1), (Wrel_2_0, Wrel_2_1))
    wroot = ((Wroot_0_0, Wroot_0_1), (Wroot_1_0, Wroot_1_1),
             (Wroot_2_0, Wroot_2_1))
    bias = ((b_0_0, b_0_1), (b_1_0, b_1_1), (b_2_0, b_2_1))

    rels1 = tuple(mp[0] for mp in METAPATHS)
    res = _make_segsum3(rels1, 1)(x, bsrc, bdst, cnt)
    sp = list(res[:3])
    dg1 = [dp.T for dp in res[3:]]
    w1 = [(wrel[i][0][mp[0]], wroot[i][0], bias[i][0].reshape(1, H))
          for i, mp in enumerate(METAPATHS)]
    h = _dense1_call(x, sp, dg1, w1)

    rels2 = tuple(mp[1] for mp in METAPATHS)
    res = _make_segsum3(rels2, 3)(h[0], h[1], h[2], bsrc, bdst, cnt)
    tp = list(res[:3])
    dg2 = [dp.T for dp in res[3:]]
    w2 = [(wrel[i][1][mp[1]], wroot[i][1], bias[i][1].reshape(1, H))
          for i, mp in enumerate(METAPATHS)]

    f1 = [fc1_W[i * H:(i + 1) * H] for i in range(3)]
    w2pad = jnp.zeros((H, H), jnp.float32).at[:, :NCLS].set(fc2_W)
    b2pad = jnp.full((1, H), -1e30, jnp.float32).at[0, :NCLS].set(fc2_b)

    return _dense2_call(h, tp, dg2, w2, f1, fc1_b.reshape(1, H), w2pad, b2pad)


# speedup vs baseline: 1.3605x; 1.0155x over previous
import jax
import jax.numpy as jnp
from jax import lax
from jax.experimental import pallas as pl
from jax.experimental.pallas import tpu as pltpu
from jax.experimental.pallas import tpu_sc as plsc

N = 10000
E = 320000
D = 128
H = 128
NUM_REL = 4
NCLS = 16
METAPATHS = ((0, 1), (2, 3), (1, 0))

NC = 2
NS = 16
NW = NC * NS
LANES = 16
CHUNK = E // NW
VPC = CHUNK // LANES
NPAD = 10240
TRASH = N
BLK = 128
NBLK = 81
CAP = NBLK * BLK
SLAB = NPAD // NS

BROWS = 1024
GRID = NPAD // BROWS


def _mesh():
    return plsc.VectorSubcoreMesh(core_axis_name="c", subcore_axis_name="s",
                                  num_cores=NC, num_subcores=NS)


_SC_PARAMS = pltpu.CompilerParams(needs_layout_passes=False,
                                  use_tc_tiling_on_sc=False)


def _wid():
    return lax.axis_index("s") * NC + lax.axis_index("c")


def _bucket_body(src_hbm, dst_hbm, type_hbm, bsrc_hbm, bdst_hbm, cnt_hbm,
                 src_v, dst_v, type_v, bsrc_v, bdst_v, cnt_v):
    wid = _wid()
    base = wid * CHUNK
    pltpu.sync_copy(src_hbm.at[pl.ds(base, CHUNK)], src_v)
    pltpu.sync_copy(dst_hbm.at[pl.ds(base, CHUNK)], dst_v)
    pltpu.sync_copy(type_hbm.at[pl.ds(base, CHUNK)], type_v)

    trash = TRASH + jnp.arange(LANES, dtype=jnp.int32) * 8
    zero = jnp.zeros((LANES,), jnp.int32)

    def prefill(i, _):
        for r in range(NUM_REL):
            bsrc_v[pl.ds(r * CAP + i * LANES, LANES)] = trash
            bdst_v[pl.ds(r * CAP + i * LANES, LANES)] = zero
        return 0

    lax.fori_loop(0, CAP // LANES, prefill, 0)

    one = jnp.ones((LANES,), jnp.int32)

    def step(i, offs):
        s = src_v[pl.ds(i * LANES, LANES)]
        d = dst_v[pl.ds(i * LANES, LANES)]
        t = type_v[pl.ds(i * LANES, LANES)]
        sh = t * 8
        cp = plsc.cumsum(jnp.left_shift(one, sh))
        rank = jnp.right_shift(cp, sh) & 255
        total = jnp.max(cp)
        offv = jnp.where(t == 0, offs[0],
                         jnp.where(t == 1, offs[1],
                                   jnp.where(t == 2, offs[2], offs[3])))
        pos = t * CAP + offv + rank - 1
        plsc.store_scatter(bsrc_v, [pos], s)
        plsc.store_scatter(bdst_v, [pos], d)
        return (offs[0] + (total & 255),
                offs[1] + (jnp.right_shift(total, 8) & 255),
                offs[2] + (jnp.right_shift(total, 16) & 255),
                offs[3] + jnp.right_shift(total, 24))

    offs = lax.fori_loop(0, VPC, step,
                         tuple(jnp.int32(0) for _ in range(NUM_REL)))
    for r in range(NUM_REL):
        cnt_v[pl.ds(r * LANES, LANES)] = zero + offs[r]
        pltpu.sync_copy(bsrc_v.at[pl.ds(r * CAP, CAP)], bsrc_hbm.at[r, wid])
        pltpu.sync_copy(bdst_v.at[pl.ds(r * CAP, CAP)], bdst_hbm.at[r, wid])
    pltpu.sync_copy(cnt_v, cnt_hbm.at[wid])


def _bucket_call(src, dst, etype):
    k = pl.kernel(
        _bucket_body,
        out_type=(
            jax.ShapeDtypeStruct((NUM_REL, NW, CAP), jnp.int32),
            jax.ShapeDtypeStruct((NUM_REL, NW, CAP), jnp.int32),
            jax.ShapeDtypeStruct((NW, NUM_REL * LANES), jnp.int32),
        ),
        mesh=_mesh(),
        compiler_params=_SC_PARAMS,
        scratch_types=[
            pltpu.VMEM((CHUNK,), jnp.int32),
            pltpu.VMEM((CHUNK,), jnp.int32),
            pltpu.VMEM((CHUNK,), jnp.int32),
            pltpu.VMEM((NUM_REL * CAP,), jnp.int32),
            pltpu.VMEM((NUM_REL * CAP,), jnp.int32),
            pltpu.VMEM((NUM_REL * LANES,), jnp.int32),
        ],
    )
    return k(src, dst, etype)


def _make_segsum3(rels, nh):
    def body(*args):
        hs = args[:nh]
        bsrc_hbm, bdst_hbm, cnt_hbm = args[nh:nh + 3]
        outs = args[nh + 3:nh + 6]
        degps = args[nh + 6:nh + 9]
        (idx_s2, idx_d2, rows_a, rows_b, cnt_v, deg_v, accum,
         sem_a, sem_b) = args[nh + 9:]

        cid = lax.axis_index("c")
        sid = lax.axis_index("s")
        wid = sid * NC + cid

        zero = jnp.zeros((LANES,), jnp.float32)
        ones = jnp.ones((LANES,), jnp.float32)
        pltpu.sync_copy(cnt_hbm.at[wid], cnt_v)

        for p, r in enumerate(rels):
            h_hbm = hs[p] if nh == 3 else hs[0]
            out_hbm = outs[p]
            degp_hbm = degps[p]

            def zfill(i, _):
                for j in range(D // LANES):
                    rows_a[i, pl.ds(j * LANES, LANES)] = zero
                return 0

            lax.fori_loop(0, BLK, zfill, 0)
            for k in range(SLAB // BLK):
                pltpu.sync_copy(rows_a,
                                accum.at[pl.ds(sid * SLAB + k * BLK, BLK)])

            def dzfill(i, _):
                deg_v[pl.ds(i * LANES, LANES)] = zero
                return 0

            lax.fori_loop(0, NPAD // LANES, dzfill, 0)
            plsc.subcore_barrier()

            n = jnp.max(cnt_v[pl.ds(r * LANES, LANES)])
            nblk = jnp.maximum((n + BLK - 1) >> 7, 1)
            ngrp = nblk >> 2

            bufs = ((rows_a, sem_a), (rows_b, sem_b))

            def start(k):
                rows, sem = bufs[k % 2]
                pltpu.async_copy(h_hbm.at[idx_d2.at[k]], rows, sem)

            def gloop(g, _):
                pltpu.sync_copy(bsrc_hbm.at[r, wid, pl.ds(g * 4, 4)], idx_s2)
                pltpu.sync_copy(bdst_hbm.at[r, wid, pl.ds(g * 4, 4)], idx_d2)
                start(0)
                for k in range(4):
                    rows, sem = bufs[k % 2]
                    pltpu.make_async_copy(h_hbm.at[idx_d2.at[k]], rows,
                                          sem).wait()
                    if k < 3:
                        start(k + 1)
                    for j in range(BLK // LANES):
                        s = idx_s2[k, pl.ds(j * LANES, LANES)]
                        plsc.addupdate_scatter(deg_v, [s], ones)
                    pltpu.sync_copy(rows, accum.at[idx_s2.at[k]], add=True)
                return 0

            lax.fori_loop(0, ngrp, gloop, 0)

            def tloop(b, _):
                pltpu.sync_copy(bsrc_hbm.at[r, wid, b], idx_s2.at[0])
                pltpu.sync_copy(bdst_hbm.at[r, wid, b], idx_d2.at[0])
                start(0)
                for j in range(BLK // LANES):
                    s = idx_s2[0, pl.ds(j * LANES, LANES)]
                    plsc.addupdate_scatter(deg_v, [s], ones)
                pltpu.make_async_copy(h_hbm.at[idx_d2.at[0]], rows_a,
                                      sem_a).wait()
                pltpu.sync_copy(rows_a, accum.at[idx_s2.at[0]], add=True)
                return 0

            lax.fori_loop(ngrp * 4, nblk, tloop, 0)

            plsc.subcore_barrier()
            for k in range(SLAB // BLK):
                sl = pl.ds(sid * SLAB + k * BLK, BLK)
                pltpu.sync_copy(accum.at[sl], out_hbm.at[cid, sl])
            pltpu.sync_copy(deg_v, degp_hbm.at[wid])

    return pl.kernel(
        body,
        out_type=(
            [jax.ShapeDtypeStruct((NC, NPAD, D), jnp.float32)] * 3
            + [jax.ShapeDtypeStruct((NW, NPAD), jnp.float32)] * 3
        ),
        mesh=_mesh(),
        compiler_params=_SC_PARAMS,
        scratch_types=[
            pltpu.VMEM((4, BLK), jnp.int32),
            pltpu.VMEM((4, BLK), jnp.int32),
            pltpu.VMEM((BLK, D), jnp.float32),
            pltpu.VMEM((BLK, D), jnp.float32),
            pltpu.VMEM((NUM_REL * LANES,), jnp.int32),
            pltpu.VMEM((NPAD,), jnp.float32),
            pltpu.VMEM_SHARED((NPAD, D), jnp.float32),
            pltpu.SemaphoreType.DMA,
            pltpu.SemaphoreType.DMA,
        ],
    )


def _dense1_body(x_ref, sp0, sp1, sp2, dg0, dg1, dg2,
                 wr0, wt0, b0, wr1, wt1, b1, wr2, wt2, b2,
                 h0, h1, h2):
    x = x_ref[...]
    ones_col = jnp.ones((NW, 1), jnp.float32)
    dn = (((0,), (0,)), ((), ()))
    for sp, dg, wr, wt, bb, out in ((sp0, dg0, wr0, wt0, b0, h0),
                                    (sp1, dg1, wr1, wt1, b1, h1),
                                    (sp2, dg2, wr2, wt2, b2, h2)):
        deg = lax.dot_general(dg[...], ones_col, dn,
                              preferred_element_type=jnp.float32)
        inv = 1.0 / jnp.maximum(deg, 1.0)
        agg = (sp[0] + sp[1]) * inv
        out[...] = jnp.maximum(
            jnp.dot(agg, wr[...], preferred_element_type=jnp.float32)
            + jnp.dot(x, wt[...], preferred_element_type=jnp.float32)
            + bb[...], 0.0)


def _dense1_call(x, sp, dg, w):
    row = pl.BlockSpec((BROWS, D), lambda i: (i, 0))
    par = pl.BlockSpec((NC, BROWS, D), lambda i: (0, i, 0))
    degs = pl.BlockSpec((NW, BROWS), lambda i: (0, i))
    mat = pl.BlockSpec((D, H), lambda i: (0, 0))
    vec = pl.BlockSpec((1, H), lambda i: (0, 0))
    return pl.pallas_call(
        _dense1_body,
        grid=(GRID,),
        in_specs=[row, par, par, par, degs, degs, degs] + [mat, mat, vec] * 3,
        out_specs=[row, row, row],
        out_shape=[jax.ShapeDtypeStruct((NPAD, H), jnp.float32)] * 3,
    )(x, sp[0], sp[1], sp[2], dg[0], dg[1], dg[2],
      w[0][0], w[0][1], w[0][2],
      w[1][0], w[1][1], w[1][2],
      w[2][0], w[2][1], w[2][2])


def _dense2_body(h0r, h1r, h2r, tp0, tp1, tp2, dg0, dg1, dg2,
                 wr0, wt0, b0, wr1, wt1, b1, wr2, wt2, b2,
                 f10, f11, f12, f1b, w2p, b2p, out):
    g = []
    ones_col = jnp.ones((NW, 1), jnp.float32)
    dn = (((0,), (0,)), ((), ()))
    for hr, tp, dg, wr, wt, bb in ((h0r, tp0, dg0, wr0, wt0, b0),
                                   (h1r, tp1, dg1, wr1, wt1, b1),
                                   (h2r, tp2, dg2, wr2, wt2, b2)):
        deg = lax.dot_general(dg[...], ones_col, dn,
                              preferred_element_type=jnp.float32)
        inv = 1.0 / jnp.maximum(deg, 1.0)
        agg = (tp[0] + tp[1]) * inv
        g.append(jnp.maximum(
            jnp.dot(agg, wr[...], preferred_element_type=jnp.float32)
            + jnp.dot(hr[...], wt[...], preferred_element_type=jnp.float32)
            + bb[...], 0.0))
    z = jnp.maximum(
        jnp.dot(g[0], f10[...], preferred_element_type=jnp.float32)
        + jnp.dot(g[1], f11[...], preferred_element_type=jnp.float32)
        + jnp.dot(g[2], f12[...], preferred_element_type=jnp.float32)
        + f1b[...], 0.0)
    logits = jnp.dot(z, w2p[...], preferred_element_type=jnp.float32) + b2p[...]
    m = jnp.max(logits, axis=1, keepdims=True)
    lse = m + jnp.log(jnp.sum(jnp.exp(logits - m), axis=1, keepdims=True))
    out[...] = (logits - lse)[:, :NCLS]


def _dense2_call(h, tp, dg, w, f1, f1b, w2p, b2p):
    row = pl.BlockSpec((BROWS, D), lambda i: (i, 0))
    par = pl.BlockSpec((NC, BROWS, D), lambda i: (0, i, 0))
    degs = pl.BlockSpec((NW, BROWS), lambda i: (0, i))
    mat = pl.BlockSpec((D, H), lambda i: (0, 0))
    vec = pl.BlockSpec((1, H), lambda i: (0, 0))
    outs = pl.BlockSpec((BROWS, NCLS), lambda i: (i, 0))
    return pl.pallas_call(
        _dense2_body,
        grid=(GRID,),
        in_specs=[row, row, row, par, par, par, degs, degs, degs]
                 + [mat, mat, vec] * 3 + [mat, mat, mat, vec, mat, vec],
        out_specs=outs,
        out_shape=jax.ShapeDtypeStruct((NPAD, NCLS), jnp.float32),
    )(h[0], h[1], h[2], tp[0], tp[1], tp[2], dg[0], dg[1], dg[2],
      w[0][0], w[0][1], w[0][2],
      w[1][0], w[1][1], w[1][2],
      w[2][0], w[2][1], w[2][2],
      f1[0], f1[1], f1[2], f1b, w2p, b2p)


def kernel(x, edge_index, edge_type,
           Wrel_0_0, Wroot_0_0, b_0_0, Wrel_0_1, Wroot_0_1, b_0_1,
           Wrel_1_0, Wroot_1_0, b_1_0, Wrel_1_1, Wroot_1_1, b_1_1,
           Wrel_2_0, Wroot_2_0, b_2_0, Wrel_2_1, Wroot_2_1, b_2_1,
           fc1_W, fc1_b, fc2_W, fc2_b):
    src = edge_index[0]
    dst = edge_index[1]
    x = jnp.concatenate([x, jnp.zeros((NPAD - N, D), jnp.float32)])

    bsrc, bdst, cnt = _bucket_call(src, dst, edge_type)
    bsrc = bsrc.reshape(NUM_REL, NW, NBLK, BLK)
    bdst = bdst.reshape(NUM_REL, NW, NBLK, BLK)
    cnt_r = [cnt[:, r * LANES:(r + 1) * LANES] for r in range(NUM_REL)]

    wrel = ((Wrel_0_0, Wrel_0_1), (Wrel_1_0, Wrel_1_1), (Wrel_2_0, Wrel_2_1))
    wroot = ((Wroot_0_0, Wroot_0_1), (Wroot_1_0, Wroot_1_1),
             (Wroot_2_0, Wroot_2_1))
    bias = ((b_0_0, b_0_1), (b_1_0, b_1_1), (b_2_0, b_2_1))

    rels1 = tuple(mp[0] for mp in METAPATHS)
    res = _make_segsum3(rels1, 1)(x, bsrc, bdst, cnt)
    sp = list(res[:3])
    dg1 = list(res[3:])
    w1 = [(wrel[i][0][mp[0]], wroot[i][0], bias[i][0].reshape(1, H))
          for i, mp in enumerate(METAPATHS)]
    h = _dense1_call(x, sp, dg1, w1)

    rels2 = tuple(mp[1] for mp in METAPATHS)
    res = _make_segsum3(rels2, 3)(h[0], h[1], h[2], bsrc, bdst, cnt)
    tp = list(res[:3])
    dg2 = list(res[3:])
    w2 = [(wrel[i][1][mp[1]], wroot[i][1], bias[i][1].reshape(1, H))
          for i, mp in enumerate(METAPATHS)]

    f1 = [fc1_W[i * H:(i + 1) * H] for i in range(3)]
    w2pad = jnp.zeros((H, H), jnp.float32).at[:, :NCLS].set(fc2_W)
    b2pad = jnp.full((1, H), -1e30, jnp.float32).at[0, :NCLS].set(fc2_b)

    out = _dense2_call(h, tp, dg2, w2, f1, fc1_b.reshape(1, H), w2pad, b2pad)
    return out[:N]
